# baseline (speedup 1.0000x reference)
import functools

import jax
import jax.numpy as jnp
from jax import lax
from jax.experimental import pallas as pl
from jax.experimental.pallas import tpu as pltpu
from jax.experimental.pallas import tpu_sc as plsc


def _make_sc_gather(B, D, NC, NS):
    NW = NC * NS
    b_per_w = B // NW
    mesh = plsc.VectorSubcoreMesh(core_axis_name="c", subcore_axis_name="s")

    @functools.partial(
        pl.kernel,
        mesh=mesh,
        out_type=(
            jax.ShapeDtypeStruct((B, D), jnp.float32),
            jax.ShapeDtypeStruct((B, D), jnp.float32),
        ),
        scratch_types=[
            pltpu.VMEM((b_per_w,), jnp.int32),
            pltpu.VMEM((b_per_w,), jnp.int32),
            pltpu.VMEM((b_per_w, D), jnp.float32),
            pltpu.VMEM((b_per_w, D), jnp.float32),
            pltpu.SemaphoreType.DMA,
            pltpu.SemaphoreType.DMA,
        ],
        compiler_params=pltpu.CompilerParams(use_tc_tiling_on_sc=False),
    )
    def sc_gather(uids_hbm, iids_hbm, utab_hbm, itab_hbm,
                  uout_hbm, iout_hbm,
                  uidx_v, iidx_v, urows_v, irows_v, usem, isem):
        wid = lax.axis_index("s") * NC + lax.axis_index("c")
        base = wid * b_per_w
        pltpu.sync_copy(uids_hbm.at[pl.ds(base, b_per_w)], uidx_v)
        pltpu.sync_copy(iids_hbm.at[pl.ds(base, b_per_w)], iidx_v)
        cu = pltpu.async_copy(utab_hbm.at[uidx_v], urows_v, usem)
        ci = pltpu.async_copy(itab_hbm.at[iidx_v], irows_v, isem)
        cu.wait()
        pltpu.sync_copy(urows_v, uout_hbm.at[pl.ds(base, b_per_w)])
        ci.wait()
        pltpu.sync_copy(irows_v, iout_hbm.at[pl.ds(base, b_per_w)])

    return sc_gather


def _mlp_body(u_ref, i_ref, w1u_ref, w1i_ref, b1_ref, w2_ref, b2_ref,
              w3_ref, b3_ref, wo_ref, bo_ref, out_ref):
    x1 = jnp.maximum(
        u_ref[...] @ w1u_ref[...] + i_ref[...] @ w1i_ref[...] + b1_ref[...],
        0.0)
    x2 = jnp.maximum(x1 @ w2_ref[...] + b2_ref[...], 0.0)
    x3 = jnp.maximum(x2 @ w3_ref[...] + b3_ref[...], 0.0)
    logits = jnp.sum(x3 * wo_ref[...], axis=1, keepdims=True) + bo_ref[...]
    out_ref[...] = jax.nn.sigmoid(logits)


def _make_tc_mlp(B, D, H1, H2, H3, BM):
    grid = (B // BM,)
    full = lambda shape: pl.BlockSpec(shape, lambda i: (0, 0))
    return pl.pallas_call(
        _mlp_body,
        grid=grid,
        in_specs=[
            pl.BlockSpec((BM, D), lambda i: (i, 0)),
            pl.BlockSpec((BM, D), lambda i: (i, 0)),
            full((D, H1)),
            full((D, H1)),
            full((1, H1)),
            full((H1, H2)),
            full((1, H2)),
            full((H2, H3)),
            full((1, H3)),
            full((1, H3)),
            full((1, 1)),
        ],
        out_specs=pl.BlockSpec((BM, 1), lambda i: (i, 0)),
        out_shape=jax.ShapeDtypeStruct((B, 1), jnp.float32),
    )


def kernel(user_ids, item_ids, user_table, item_table,
           W1, b1, W2, b2, W3, b3, Wo, bo):
    B = user_ids.shape[0]
    D = user_table.shape[1]
    H1, H2, H3 = W1.shape[1], W2.shape[1], W3.shape[1]

    info = plsc.get_sparse_core_info()
    NC, NS = info.num_cores, info.num_subcores

    sc_gather = _make_sc_gather(B, D, NC, NS)
    u_emb, i_emb = sc_gather(
        user_ids.astype(jnp.int32), item_ids.astype(jnp.int32),
        user_table, item_table)

    tc_mlp = _make_tc_mlp(B, D, H1, H2, H3, BM=2048)
    out = tc_mlp(
        u_emb, i_emb,
        W1[:D, :], W1[D:, :], b1.reshape(1, H1),
        W2, b2.reshape(1, H2),
        W3, b3.reshape(1, H3),
        Wo.reshape(1, H3), bo.reshape(1, 1))
    return out


# speedup vs baseline: 3.5699x; 3.5699x over previous
import functools

import jax
import jax.numpy as jnp
from jax import lax
from jax.experimental import pallas as pl
from jax.experimental.pallas import tpu as pltpu
from jax.experimental.pallas import tpu_sc as plsc

_NBUF = 8


def _make_sc_gather(B, D, NC, NS):
    NW = NC * NS
    n_per_w = B // NW
    mesh = plsc.VectorSubcoreMesh(core_axis_name="c", subcore_axis_name="s")

    @functools.partial(
        pl.kernel,
        mesh=mesh,
        out_type=(
            jax.ShapeDtypeStruct((B * D,), jnp.float32),
            jax.ShapeDtypeStruct((B * D,), jnp.float32),
        ),
        scratch_types=[
            pltpu.VMEM((n_per_w,), jnp.int32),
            pltpu.VMEM((n_per_w,), jnp.int32),
            [pltpu.VMEM((D, 128), jnp.float32) for _ in range(_NBUF)],
            [pltpu.VMEM((D, 128), jnp.float32) for _ in range(_NBUF)],
            pltpu.VMEM((n_per_w * D,), jnp.float32),
            pltpu.VMEM((n_per_w * D,), jnp.float32),
            [pltpu.SemaphoreType.DMA for _ in range(_NBUF)],
            [pltpu.SemaphoreType.DMA for _ in range(_NBUF)],
        ],
        compiler_params=pltpu.CompilerParams(
            use_tc_tiling_on_sc=True, needs_layout_passes=False),
    )
    def sc_gather(uids_hbm, iids_hbm, utabT_hbm, itabT_hbm,
                  uout_hbm, iout_hbm,
                  uidx_v, iidx_v, ubufs, ibufs,
                  urows, irows, usems, isems):
        wid = lax.axis_index("s") * NC + lax.axis_index("c")
        base = wid * n_per_w
        pltpu.sync_copy(uids_hbm.at[pl.ds(base, n_per_w)], uidx_v)
        pltpu.sync_copy(iids_hbm.at[pl.ds(base, n_per_w)], iidx_v)

        rows_lo = lax.iota(jnp.int32, 16)
        rows_hi = rows_lo + 16

        def issue(idx_scalar, tab, bufs, sems, slot):
            c = pl.multiple_of((idx_scalar >> 7) << 7, 128)
            pltpu.async_copy(tab.at[:, pl.ds(c, 128)], bufs[slot], sems[slot])

        def extract(k, lane_scalar, bufs, rows, slot):
            lane = jnp.full((16,), lane_scalar & 127, jnp.int32)
            g0 = plsc.load_gather(bufs[slot], [rows_lo, lane])
            g1 = plsc.load_gather(bufs[slot], [rows_hi, lane])
            rows[pl.ds(k * D, 16)] = g0
            rows[pl.ds(k * D + 16, 16)] = g1

        zeros16 = jnp.zeros((16,), jnp.int32)

        def body(t, carry):
            uv_prev, iv_prev = carry
            uv = uidx_v[pl.ds(t * 16, 16)]
            iv = iidx_v[pl.ds(t * 16, 16)]
            for j in range(16):
                slot = j % _NBUF
                k_old = t * 16 + j - _NBUF
                u_old = uv_prev[j + 16 - _NBUF] if j < _NBUF else uv[j - _NBUF]
                i_old = iv_prev[j + 16 - _NBUF] if j < _NBUF else iv[j - _NBUF]

                @pl.when(k_old >= 0)
                def _():
                    pltpu.make_async_copy(
                        utabT_hbm.at[:, pl.ds(0, 128)], ubufs[slot],
                        usems[slot]).wait()
                    extract(k_old, u_old, ubufs, urows, slot)
                    pltpu.make_async_copy(
                        itabT_hbm.at[:, pl.ds(0, 128)], ibufs[slot],
                        isems[slot]).wait()
                    extract(k_old, i_old, ibufs, irows, slot)

                issue(uv[j], utabT_hbm, ubufs, usems, slot)
                issue(iv[j], itabT_hbm, ibufs, isems, slot)
            return (uv, iv)

        uv, iv = lax.fori_loop(
            0, n_per_w // 16, body, (zeros16, zeros16))

        for j in range(_NBUF):
            slot = j % _NBUF
            k_old = n_per_w - _NBUF + j
            pltpu.make_async_copy(
                utabT_hbm.at[:, pl.ds(0, 128)], ubufs[slot],
                usems[slot]).wait()
            extract(k_old, uv[16 - _NBUF + j], ubufs, urows, slot)
            pltpu.make_async_copy(
                itabT_hbm.at[:, pl.ds(0, 128)], ibufs[slot],
                isems[slot]).wait()
            extract(k_old, iv[16 - _NBUF + j], ibufs, irows, slot)

        pltpu.sync_copy(urows, uout_hbm.at[pl.ds(base * D, n_per_w * D)])
        pltpu.sync_copy(irows, iout_hbm.at[pl.ds(base * D, n_per_w * D)])

    return sc_gather


def _mlp_body(u_ref, i_ref, w1u_ref, w1i_ref, b1_ref, w2_ref, b2_ref,
              w3_ref, b3_ref, wo_ref, bo_ref, out_ref):
    x1 = jnp.maximum(
        u_ref[...] @ w1u_ref[...] + i_ref[...] @ w1i_ref[...] + b1_ref[...],
        0.0)
    x2 = jnp.maximum(x1 @ w2_ref[...] + b2_ref[...], 0.0)
    x3 = jnp.maximum(x2 @ w3_ref[...] + b3_ref[...], 0.0)
    logits = jnp.sum(x3 * wo_ref[...], axis=1, keepdims=True) + bo_ref[...]
    out_ref[...] = jax.nn.sigmoid(logits)


def _make_tc_mlp(B, D, H1, H2, H3, BM):
    grid = (B // BM,)
    full = lambda shape: pl.BlockSpec(shape, lambda i: (0, 0))
    return pl.pallas_call(
        _mlp_body,
        grid=grid,
        in_specs=[
            pl.BlockSpec((BM, D), lambda i: (i, 0)),
            pl.BlockSpec((BM, D), lambda i: (i, 0)),
            full((D, H1)),
            full((D, H1)),
            full((1, H1)),
            full((H1, H2)),
            full((1, H2)),
            full((H2, H3)),
            full((1, H3)),
            full((1, H3)),
            full((1, 1)),
        ],
        out_specs=pl.BlockSpec((BM, 1), lambda i: (i, 0)),
        out_shape=jax.ShapeDtypeStruct((B, 1), jnp.float32),
    )


def kernel(user_ids, item_ids, user_table, item_table,
           W1, b1, W2, b2, W3, b3, Wo, bo):
    B = user_ids.shape[0]
    D = user_table.shape[1]
    H1, H2, H3 = W1.shape[1], W2.shape[1], W3.shape[1]

    info = plsc.get_sparse_core_info()
    NC, NS = info.num_cores, info.num_subcores

    sc_gather = _make_sc_gather(B, D, NC, NS)
    u_flat, i_flat = sc_gather(
        user_ids.astype(jnp.int32), item_ids.astype(jnp.int32),
        user_table.T, item_table.T)
    u_emb = u_flat.reshape(B, D)
    i_emb = i_flat.reshape(B, D)

    tc_mlp = _make_tc_mlp(B, D, H1, H2, H3, BM=2048)
    out = tc_mlp(
        u_emb, i_emb,
        W1[:D, :], W1[D:, :], b1.reshape(1, H1),
        W2, b2.reshape(1, H2),
        W3, b3.reshape(1, H3),
        Wo.reshape(1, H3), bo.reshape(1, 1))
    return out


# speedup vs baseline: 3.8611x; 1.0816x over previous
import functools

import jax
import jax.numpy as jnp
from jax import lax
from jax.experimental import pallas as pl
from jax.experimental.pallas import tpu as pltpu
from jax.experimental.pallas import tpu_sc as plsc

_NS_ = 8


def _make_sc_gather(B, D, NC, NS):
    NW = NC * NS
    n_per_w = B // NW
    n_blocks = n_per_w // 16
    mesh = plsc.VectorSubcoreMesh(core_axis_name="c", subcore_axis_name="s")

    @functools.partial(
        pl.kernel,
        mesh=mesh,
        out_type=(
            jax.ShapeDtypeStruct((D, B), jnp.float32),
            jax.ShapeDtypeStruct((D, B), jnp.float32),
        ),
        scratch_types=[
            pltpu.VMEM((n_per_w,), jnp.int32),
            pltpu.VMEM((n_per_w,), jnp.int32),
            pltpu.VMEM((_NS_, D, 128), jnp.float32),
            pltpu.VMEM((_NS_, D, 128), jnp.float32),
            pltpu.VMEM((D, n_per_w), jnp.float32),
            pltpu.VMEM((D, n_per_w), jnp.float32),
            [pltpu.SemaphoreType.DMA for _ in range(_NS_)],
            [pltpu.SemaphoreType.DMA for _ in range(_NS_)],
        ],
        compiler_params=pltpu.CompilerParams(
            use_tc_tiling_on_sc=True, needs_layout_passes=False),
    )
    def sc_gather(uids_hbm, iids_hbm, utabT_hbm, itabT_hbm,
                  uout_hbm, iout_hbm,
                  uidx_v, iidx_v, ubufs, ibufs,
                  urowsT, irowsT, usems, isems):
        wid = lax.axis_index("s") * NC + lax.axis_index("c")
        base = wid * n_per_w
        pltpu.sync_copy(uids_hbm.at[pl.ds(base, n_per_w)], uidx_v)
        pltpu.sync_copy(iids_hbm.at[pl.ds(base, n_per_w)], iidx_v)

        iota16 = lax.iota(jnp.int32, 16)
        slot_vec = iota16 & 7
        half_sel = iota16 >> 3

        def issue(vv, j, tab, bufs, sems):
            c = pl.multiple_of((vv[j] >> 7) << 7, 128)
            pltpu.async_copy(
                tab.at[:, pl.ds(c, 128)], bufs.at[j & 7], sems[j & 7])

        def wait_all(tab, bufs, sems):
            for s in range(_NS_):
                pltpu.make_async_copy(
                    tab.at[:, pl.ds(0, 128)], bufs.at[s], sems[s]).wait()

        def extract(k0, idx_v, bufs, rowsT, tab, sems):
            wait_all(tab, bufs, sems)
            lanes = plsc.load_gather(idx_v, [k0 + slot_vec]) & 127
            cols = k0 + slot_vec
            for rp in range(D // 2):
                row_vec = 2 * rp + half_sel
                g = plsc.load_gather(bufs, [slot_vec, row_vec, lanes])
                plsc.store_scatter(rowsT, [row_vec, cols], g)

        def body(t, _c):
            uv_n = uidx_v[pl.ds(t * 16, 16)]
            iv_n = iidx_v[pl.ds(t * 16, 16)]
            for h in (0, 1):
                cond = (t > 0) if h == 0 else (t >= 0)
                k_old = t * 16 + h * 8 - 8

                @pl.when(cond)
                def _():
                    extract(k_old, uidx_v, ubufs, urowsT, utabT_hbm, usems)

                for j in range(h * 8, h * 8 + 8):
                    issue(uv_n, j, utabT_hbm, ubufs, usems)

                @pl.when(cond)
                def _():
                    extract(k_old, iidx_v, ibufs, irowsT, itabT_hbm, isems)

                for j in range(h * 8, h * 8 + 8):
                    issue(iv_n, j, itabT_hbm, ibufs, isems)
            return 0

        lax.fori_loop(0, n_blocks, body, 0)

        k_last = n_per_w - 8
        extract(k_last, uidx_v, ubufs, urowsT, utabT_hbm, usems)
        extract(k_last, iidx_v, ibufs, irowsT, itabT_hbm, isems)

        pltpu.sync_copy(urowsT, uout_hbm.at[:, pl.ds(base, n_per_w)])
        pltpu.sync_copy(irowsT, iout_hbm.at[:, pl.ds(base, n_per_w)])

    return sc_gather


def _mlp_body(uT_ref, iT_ref, w1u_ref, w1i_ref, b1_ref, w2_ref, b2_ref,
              w3_ref, b3_ref, wo_ref, bo_ref, out_ref):
    dn = (((0,), (0,)), ((), ()))
    x1 = jnp.maximum(
        lax.dot_general(w1u_ref[...], uT_ref[...], dn)
        + lax.dot_general(w1i_ref[...], iT_ref[...], dn)
        + b1_ref[...],
        0.0)
    x2 = jnp.maximum(lax.dot_general(w2_ref[...], x1, dn) + b2_ref[...], 0.0)
    x3 = jnp.maximum(lax.dot_general(w3_ref[...], x2, dn) + b3_ref[...], 0.0)
    logits = lax.dot_general(wo_ref[...], x3, dn) + bo_ref[...]
    out_ref[...] = jax.nn.sigmoid(logits)


def _make_tc_mlp(B, D, H1, H2, H3, BM):
    grid = (B // BM,)
    full = lambda shape: pl.BlockSpec(shape, lambda i: (0, 0))
    return pl.pallas_call(
        _mlp_body,
        grid=grid,
        in_specs=[
            pl.BlockSpec((D, BM), lambda i: (0, i)),
            pl.BlockSpec((D, BM), lambda i: (0, i)),
            full((D, H1)),
            full((D, H1)),
            full((H1, 1)),
            full((H1, H2)),
            full((H2, 1)),
            full((H2, H3)),
            full((H3, 1)),
            full((H3, 1)),
            full((1, 1)),
        ],
        out_specs=pl.BlockSpec((1, BM), lambda i: (0, i)),
        out_shape=jax.ShapeDtypeStruct((1, B), jnp.float32),
    )


def kernel(user_ids, item_ids, user_table, item_table,
           W1, b1, W2, b2, W3, b3, Wo, bo):
    B = user_ids.shape[0]
    D = user_table.shape[1]
    H1, H2, H3 = W1.shape[1], W2.shape[1], W3.shape[1]

    info = plsc.get_sparse_core_info()
    NC, NS = info.num_cores, info.num_subcores

    sc_gather = _make_sc_gather(B, D, NC, NS)
    u_embT, i_embT = sc_gather(
        user_ids.astype(jnp.int32), item_ids.astype(jnp.int32),
        user_table.T, item_table.T)

    tc_mlp = _make_tc_mlp(B, D, H1, H2, H3, BM=2048)
    out = tc_mlp(
        u_embT, i_embT,
        W1[:D, :], W1[D:, :], b1.reshape(H1, 1),
        W2, b2.reshape(H2, 1),
        W3, b3.reshape(H3, 1),
        Wo, bo.reshape(1, 1))
    return out.reshape(B, 1)


# speedup vs baseline: 3.8895x; 1.0074x over previous
import functools

import jax
import jax.numpy as jnp
from jax import lax
from jax.experimental import pallas as pl
from jax.experimental.pallas import tpu as pltpu
from jax.experimental.pallas import tpu_sc as plsc

_NS_ = 8


def _make_sc_gather(B, D, NC, NS):
    NW = NC * NS
    n_per_w = B // NW
    n_blocks = n_per_w // 16
    mesh = plsc.VectorSubcoreMesh(core_axis_name="c", subcore_axis_name="s")

    @functools.partial(
        pl.kernel,
        mesh=mesh,
        out_type=(
            jax.ShapeDtypeStruct((D, B), jnp.float32),
            jax.ShapeDtypeStruct((D, B), jnp.float32),
        ),
        scratch_types=[
            pltpu.VMEM((n_per_w,), jnp.int32),
            pltpu.VMEM((n_per_w,), jnp.int32),
            pltpu.VMEM((_NS_, D, 128), jnp.float32),
            pltpu.VMEM((_NS_, D, 128), jnp.float32),
            pltpu.VMEM((D, n_per_w), jnp.float32),
            pltpu.VMEM((D, n_per_w), jnp.float32),
            [pltpu.SemaphoreType.DMA for _ in range(_NS_)],
            [pltpu.SemaphoreType.DMA for _ in range(_NS_)],
        ],
        compiler_params=pltpu.CompilerParams(
            use_tc_tiling_on_sc=True, needs_layout_passes=False),
    )
    def sc_gather(uids_hbm, iids_hbm, utabT_hbm, itabT_hbm,
                  uout_hbm, iout_hbm,
                  uidx_v, iidx_v, ubufs, ibufs,
                  urowsT, irowsT, usems, isems):
        wid = lax.axis_index("s") * NC + lax.axis_index("c")
        base = wid * n_per_w
        pltpu.sync_copy(uids_hbm.at[pl.ds(base, n_per_w)], uidx_v)
        pltpu.sync_copy(iids_hbm.at[pl.ds(base, n_per_w)], iidx_v)

        iota16 = lax.iota(jnp.int32, 16)
        slot_vec = iota16 & 7
        half_sel = iota16 >> 3

        def issue(vv, j, tab, bufs, sems):
            c = pl.multiple_of((vv[j] >> 7) << 7, 128)
            pltpu.async_copy(
                tab.at[:, pl.ds(c, 128)], bufs.at[j & 7], sems[j & 7])

        def wait_all(tab, bufs, sems):
            for s in range(_NS_):
                pltpu.make_async_copy(
                    tab.at[:, pl.ds(0, 128)], bufs.at[s], sems[s]).wait()

        def extract(k0, idx_v, bufs, rowsT, tab, sems):
            wait_all(tab, bufs, sems)
            lanes = plsc.load_gather(idx_v, [k0 + slot_vec]) & 127
            cols = k0 + slot_vec
            for rp in range(D // 2):
                row_vec = 2 * rp + half_sel
                g = plsc.load_gather(bufs, [slot_vec, row_vec, lanes])
                plsc.store_scatter(rowsT, [row_vec, cols], g)

        def body(t, _c):
            uv_n = uidx_v[pl.ds(t * 16, 16)]
            iv_n = iidx_v[pl.ds(t * 16, 16)]
            for h in (0, 1):
                cond = (t > 0) if h == 0 else (t >= 0)
                k_old = t * 16 + h * 8 - 8

                @pl.when(cond)
                def _():
                    extract(k_old, uidx_v, ubufs, urowsT, utabT_hbm, usems)

                for j in range(h * 8, h * 8 + 8):
                    issue(uv_n, j, utabT_hbm, ubufs, usems)

                @pl.when(cond)
                def _():
                    extract(k_old, iidx_v, ibufs, irowsT, itabT_hbm, isems)

                for j in range(h * 8, h * 8 + 8):
                    issue(iv_n, j, itabT_hbm, ibufs, isems)
            return 0

        lax.fori_loop(0, n_blocks, body, 0)

        k_last = n_per_w - 8
        extract(k_last, uidx_v, ubufs, urowsT, utabT_hbm, usems)
        extract(k_last, iidx_v, ibufs, irowsT, itabT_hbm, isems)

        pltpu.sync_copy(urowsT, uout_hbm.at[:, pl.ds(base, n_per_w)])
        pltpu.sync_copy(irowsT, iout_hbm.at[:, pl.ds(base, n_per_w)])

    return sc_gather


def _mlp_body(uT_ref, iT_ref, w1u_ref, w1i_ref, b1_ref, w2_ref, b2_ref,
              w3_ref, b3_ref, wo_ref, bo_ref, out_ref):
    dn = (((0,), (0,)), ((), ()))
    x1 = jnp.maximum(
        lax.dot_general(w1u_ref[...], uT_ref[...], dn)
        + lax.dot_general(w1i_ref[...], iT_ref[...], dn)
        + b1_ref[...],
        0.0)
    x2 = jnp.maximum(lax.dot_general(w2_ref[...], x1, dn) + b2_ref[...], 0.0)
    x3 = jnp.maximum(lax.dot_general(w3_ref[...], x2, dn) + b3_ref[...], 0.0)
    logits = lax.dot_general(wo_ref[...], x3, dn) + bo_ref[...]
    out_ref[...] = jax.nn.sigmoid(logits)


def _make_tc_mlp(B, D, H1, H2, H3, BM):
    grid = (B // BM,)
    full = lambda shape: pl.BlockSpec(shape, lambda i: (0, 0))
    return pl.pallas_call(
        _mlp_body,
        grid=grid,
        in_specs=[
            pl.BlockSpec((D, BM), lambda i: (0, i)),
            pl.BlockSpec((D, BM), lambda i: (0, i)),
            full((D, H1)),
            full((D, H1)),
            full((H1, 1)),
            full((H1, H2)),
            full((H2, 1)),
            full((H2, H3)),
            full((H3, 1)),
            full((H3, 1)),
            full((1, 1)),
        ],
        out_specs=pl.BlockSpec((1, BM), lambda i: (0, i)),
        out_shape=jax.ShapeDtypeStruct((1, B), jnp.float32),
    )


def kernel(user_ids, item_ids, user_table, item_table,
           W1, b1, W2, b2, W3, b3, Wo, bo):
    B = user_ids.shape[0]
    D = user_table.shape[1]
    H1, H2, H3 = W1.shape[1], W2.shape[1], W3.shape[1]

    info = plsc.get_sparse_core_info()
    NC, NS = info.num_cores, info.num_subcores

    sc_gather = _make_sc_gather(B, D, NC, NS)
    u_embT, i_embT = sc_gather(
        user_ids.astype(jnp.int32), item_ids.astype(jnp.int32),
        user_table.T, item_table.T)

    tc_mlp = _make_tc_mlp(B, D, H1, H2, H3, BM=4096)
    out = tc_mlp(
        u_embT, i_embT,
        W1[:D, :], W1[D:, :], b1.reshape(H1, 1),
        W2, b2.reshape(H2, 1),
        W3, b3.reshape(H3, 1),
        Wo, bo.reshape(1, 1))
    return out.reshape(B, 1)
